# Initial kernel scaffold; baseline (speedup 1.0000x reference)
import jax
import jax.numpy as jnp
from jax.experimental import pallas as pl


def kernel(x, edge_index, edge_attr, batch, edge_batch, nn_W, nn_b, root, conv_bias, bn_gamma, bn_beta, s_W, s_b, t_W, t_b):
    raise NotImplementedError("write your pallas kernel here")


# speedup vs baseline: 4.0143x; 4.0143x over previous
import functools

import numpy as np
import jax
import jax.numpy as jnp
from jax import lax
from jax.experimental import pallas as pl
from jax.experimental.pallas import tpu as pltpu
from jax.experimental.pallas import tpu_sc as plsc

N = 10000
E = 160000
IN = 64
HID = 32
EDGE = 16
OUT = 128
G = 64

KP1 = EDGE + 1
KW = KP1 * HID

NC = 2
NS = 16
NW = NC * NS
M = 128
EP = 163840
ROWS = EP // M
RPW = ROWS // NW
EPW = EP // NW
SUB = 4
CH = SUB * M
NCH = RPW // SUB
NA = 10240
NPS = NA // NS

_MESH = dict(core_axis_name="c", subcore_axis_name="s")

_R_NP = np.zeros((KP1, KW), np.float32)
for _k in range(KP1):
    _R_NP[_k, _k * HID:(_k + 1) * HID] = 1.0
_S_NP = np.zeros((KW, HID), np.float32)
for _k in range(KP1):
    _S_NP[_k * HID + np.arange(HID), np.arange(HID)] = 1.0


def _sc_gather_body(x_hbm, src_hbm, xj_hbm, idx_v, rows_v, sem):
    wid = lax.axis_index("s") * NC + lax.axis_index("c")
    row0 = wid * RPW
    base = wid * EPW
    pltpu.sync_copy(src_hbm.at[pl.ds(row0, RPW)], idx_v)

    def step(c, carry):
        cps = []
        for j in range(SUB):
            cps.append(pltpu.async_copy(
                x_hbm.at[idx_v.at[c * SUB + j]],
                rows_v.at[pl.ds(j * M, M)], sem))
        for cp in cps:
            cp.wait()
        pltpu.sync_copy(rows_v, xj_hbm.at[pl.ds(base + c * CH, CH)])
        return carry

    lax.fori_loop(0, NCH, step, 0)


@functools.cache
def _make_sc_gather():
    return functools.partial(
        pl.kernel,
        mesh=plsc.VectorSubcoreMesh(**_MESH),
        out_type=jax.ShapeDtypeStruct((EP, IN), jnp.float32),
        scratch_types=[
            pltpu.VMEM((RPW, M), jnp.int32),
            pltpu.VMEM((CH, IN), jnp.float32),
            pltpu.SemaphoreType.DMA,
        ],
        compiler_params=pltpu.CompilerParams(use_tc_tiling_on_sc=False),
    )(_sc_gather_body)


def _sc_scatter_body(msg_hbm, dst_hbm, zeros_hbm, out_hbm,
                     idx_v, msg_v, buf_v, acc_sh, sem):
    cid = lax.axis_index("c")
    sid = lax.axis_index("s")
    wid = sid * NC + cid
    row0 = wid * RPW
    base = wid * EPW

    pltpu.sync_copy(zeros_hbm, buf_v)
    pltpu.sync_copy(buf_v, acc_sh.at[pl.ds(sid * NPS, NPS)])
    pltpu.sync_copy(dst_hbm.at[pl.ds(row0, RPW)], idx_v)
    plsc.subcore_barrier()

    def step(c, carry):
        pltpu.sync_copy(msg_hbm.at[pl.ds(base + c * CH, CH)], msg_v)
        for j in range(SUB):
            pltpu.sync_copy(msg_v.at[pl.ds(j * M, M)],
                            acc_sh.at[idx_v.at[c * SUB + j]], add=True)
        return carry

    lax.fori_loop(0, NCH, step, 0)
    plsc.subcore_barrier()

    pltpu.sync_copy(acc_sh.at[pl.ds(sid * NPS, NPS)], buf_v)
    pltpu.sync_copy(buf_v, out_hbm.at[pl.ds(cid * NA + sid * NPS, NPS)])


@functools.cache
def _make_sc_scatter():
    return functools.partial(
        pl.kernel,
        mesh=plsc.VectorSubcoreMesh(**_MESH),
        out_type=jax.ShapeDtypeStruct((NC * NA, HID), jnp.float32),
        scratch_types=[
            pltpu.VMEM((RPW, M), jnp.int32),
            pltpu.VMEM((CH, HID), jnp.float32),
            pltpu.VMEM((NPS, HID), jnp.float32),
            pltpu.VMEM_SHARED((NA, HID), jnp.float32),
            pltpu.SemaphoreType.DMA,
        ],
        compiler_params=pltpu.CompilerParams(use_tc_tiling_on_sc=False),
    )(_sc_scatter_body)


BE = 4096


def _msg_body(xj_ref, ea_ref, wf_ref, r_ref, out_ref):
    t = jnp.dot(xj_ref[...], wf_ref[...], preferred_element_type=jnp.float32)
    u = jnp.dot(ea_ref[...], r_ref[...], preferred_element_type=jnp.float32)
    v = t * u
    a = v[:, 0:128] + v[:, 128:256] + v[:, 256:384] + v[:, 384:512]
    b = a[:, 0:64] + a[:, 64:128]
    out_ref[...] = b[:, 0:32] + b[:, 32:64] + v[:, 512:544]


def _tc_msg(x_j, ea_aug, wfull, r_m):
    return pl.pallas_call(
        _msg_body,
        grid=(EP // BE,),
        in_specs=[
            pl.BlockSpec((BE, IN), lambda i: (i, 0)),
            pl.BlockSpec((BE, KP1), lambda i: (i, 0)),
            pl.BlockSpec((IN, KW), lambda i: (0, 0)),
            pl.BlockSpec((KP1, KW), lambda i: (0, 0)),
        ],
        out_specs=pl.BlockSpec((BE, HID), lambda i: (i, 0)),
        out_shape=jax.ShapeDtypeStruct((EP, HID), jnp.float32),
    )(x_j, ea_aug, wfull, r_m)


def _epi_body(p0_ref, p1_ref, x_ref, b_ref, root_ref, cb_ref, g_ref, be_ref,
              sw_ref, sb_ref, tw_ref, tb_ref, out_ref):
    h = (p0_ref[...] + p1_ref[...]
         + jnp.dot(x_ref[...], root_ref[...],
                   preferred_element_type=jnp.float32)
         + cb_ref[...])
    mean = jnp.mean(h, axis=0, keepdims=True)
    var = jnp.mean(jnp.square(h - mean), axis=0, keepdims=True)
    hn = (h - mean) * lax.rsqrt(var + 1e-5) * g_ref[...] + be_ref[...]
    hn = jnp.maximum(hn, 0.0)
    s_x = jnp.dot(hn, sw_ref[...],
                  preferred_element_type=jnp.float32) + sb_ref[...]
    t_x = jnp.dot(hn, tw_ref[...],
                  preferred_element_type=jnp.float32) + tb_ref[...]
    s_x = jnp.clip(s_x, -30.0, 30.0)
    f_x = jnp.tanh(t_x) / (1.0 + jnp.exp(s_x))
    gid = lax.broadcasted_iota(jnp.int32, (N, G), 1)
    mask = (b_ref[...] == gid).astype(jnp.float32)
    out_ref[...] = lax.dot_general(mask, f_x, (((0,), (0,)), ((), ())),
                                   preferred_element_type=jnp.float32)


def _tc_epilogue(p0, p1, x, batch2d, root, cb, gamma, beta, sw, sb, tw, tb):
    full = lambda s: pl.BlockSpec(s, lambda: (0,) * len(s))
    return pl.pallas_call(
        _epi_body,
        in_specs=[
            full((N, HID)), full((N, HID)), full((N, IN)), full((N, 1)),
            full((IN, HID)), full((1, HID)), full((1, HID)), full((1, HID)),
            full((HID, OUT)), full((1, OUT)), full((HID, OUT)),
            full((1, OUT)),
        ],
        out_specs=full((G, OUT)),
        out_shape=jax.ShapeDtypeStruct((G, OUT), jnp.float32),
    )(p0, p1, x, batch2d, root, cb, gamma, beta, sw, sb, tw, tb)


def kernel(x, edge_index, edge_attr, batch, edge_batch,
           nn_W, nn_b, root, conv_bias, bn_gamma, bn_beta,
           s_W, s_b, t_W, t_b):
    pad = EP - E
    src = jnp.concatenate(
        [edge_index[0].astype(jnp.int32), jnp.zeros((pad,), jnp.int32)]
    ).reshape(ROWS, M)
    dst = jnp.concatenate(
        [edge_index[1].astype(jnp.int32), jnp.full((pad,), N, jnp.int32)]
    ).reshape(ROWS, M)

    x_j = _make_sc_gather()(x, src)

    ea_aug = jnp.concatenate(
        [edge_attr, jnp.ones((E, 1), jnp.float32)], axis=1)
    ea_aug = jnp.concatenate(
        [ea_aug, jnp.zeros((pad, KP1), jnp.float32)], axis=0)
    wfull = jnp.concatenate(
        [nn_W.reshape(IN, HID, EDGE).transpose(0, 2, 1).reshape(IN, EDGE * HID),
         nn_b.reshape(IN, HID)], axis=1)
    msg = _tc_msg(x_j, ea_aug, wfull, jnp.asarray(_R_NP))

    partials = _make_sc_scatter()(
        msg, dst, jnp.zeros((NPS, HID), jnp.float32))

    out = _tc_epilogue(
        partials[:N], partials[NA:NA + N], x,
        batch.astype(jnp.int32).reshape(N, 1),
        root, conv_bias.reshape(1, HID),
        bn_gamma.reshape(1, HID), bn_beta.reshape(1, HID),
        s_W.T, s_b.reshape(1, OUT), t_W.T, t_b.reshape(1, OUT))
    return out


# speedup vs baseline: 4.2184x; 1.0509x over previous
import functools

import numpy as np
import jax
import jax.numpy as jnp
from jax import lax
from jax.experimental import pallas as pl
from jax.experimental.pallas import tpu as pltpu
from jax.experimental.pallas import tpu_sc as plsc

N = 10000
E = 160000
IN = 64
HID = 32
EDGE = 16
OUT = 128
G = 64

KP1 = EDGE + 1
KW = KP1 * HID

NC = 2
NS = 16
NW = NC * NS
M = 128
EP = 163840
ROWS = EP // M
RPW = ROWS // NW
EPW = EP // NW
SUB = 4
CH = SUB * M
NCH = RPW // SUB
NA = 10240
NPS = NA // NS

_MESH = dict(core_axis_name="c", subcore_axis_name="s")

_R_NP = np.zeros((KP1, KW), np.float32)
for _k in range(KP1):
    _R_NP[_k, _k * HID:(_k + 1) * HID] = 1.0
_S_NP = np.zeros((KW, HID), np.float32)
for _k in range(KP1):
    _S_NP[_k * HID + np.arange(HID), np.arange(HID)] = 1.0


def _sc_gather_body(x_hbm, src_hbm, xj_hbm, idx_v, rows_a, rows_b,
                    gsem, wsem_a, wsem_b):
    wid = lax.axis_index("s") * NC + lax.axis_index("c")
    row0 = wid * RPW
    base = wid * EPW
    pltpu.sync_copy(src_hbm.at[pl.ds(row0, RPW)], idx_v)
    bufs = ((rows_a, wsem_a), (rows_b, wsem_b))

    def step(c2, carry):
        for b in range(2):
            c = 2 * c2 + b
            buf, wsem = bufs[b]

            @pl.when(c2 > 0)
            def _():
                pltpu.make_async_copy(
                    buf, xj_hbm.at[pl.ds(base, CH)], wsem).wait()

            cps = []
            for j in range(SUB):
                cps.append(pltpu.async_copy(
                    x_hbm.at[idx_v.at[c * SUB + j]],
                    buf.at[pl.ds(j * M, M)], gsem))
            for cp in cps:
                cp.wait()
            pltpu.async_copy(buf, xj_hbm.at[pl.ds(base + c * CH, CH)], wsem)
        return carry

    lax.fori_loop(0, NCH // 2, step, 0)
    for b in range(2):
        buf, wsem = bufs[b]
        pltpu.make_async_copy(
            buf, xj_hbm.at[pl.ds(base, CH)], wsem).wait()


@functools.cache
def _make_sc_gather():
    return functools.partial(
        pl.kernel,
        mesh=plsc.VectorSubcoreMesh(**_MESH),
        out_type=jax.ShapeDtypeStruct((EP, IN), jnp.bfloat16),
        scratch_types=[
            pltpu.VMEM((RPW, M), jnp.int32),
            pltpu.VMEM((CH, IN), jnp.bfloat16),
            pltpu.VMEM((CH, IN), jnp.bfloat16),
            pltpu.SemaphoreType.DMA,
            pltpu.SemaphoreType.DMA,
            pltpu.SemaphoreType.DMA,
        ],
        compiler_params=pltpu.CompilerParams(use_tc_tiling_on_sc=False),
    )(_sc_gather_body)


def _sc_scatter_body(msg_hbm, dst_hbm, zeros_hbm, out_hbm,
                     idx_v, msg_v, buf_v, acc_sh, sem):
    cid = lax.axis_index("c")
    sid = lax.axis_index("s")
    wid = sid * NC + cid
    row0 = wid * RPW
    base = wid * EPW

    pltpu.sync_copy(zeros_hbm, buf_v)
    pltpu.sync_copy(buf_v, acc_sh.at[pl.ds(sid * NPS, NPS)])
    pltpu.sync_copy(dst_hbm.at[pl.ds(row0, RPW)], idx_v)
    plsc.subcore_barrier()

    def step(c, carry):
        pltpu.sync_copy(msg_hbm.at[pl.ds(base + c * CH, CH)], msg_v)
        for j in range(SUB):
            pltpu.sync_copy(msg_v.at[pl.ds(j * M, M)],
                            acc_sh.at[idx_v.at[c * SUB + j]], add=True)
        return carry

    lax.fori_loop(0, NCH, step, 0)
    plsc.subcore_barrier()

    pltpu.sync_copy(acc_sh.at[pl.ds(sid * NPS, NPS)], buf_v)
    pltpu.sync_copy(buf_v, out_hbm.at[pl.ds(cid * NA + sid * NPS, NPS)])


@functools.cache
def _make_sc_scatter():
    return functools.partial(
        pl.kernel,
        mesh=plsc.VectorSubcoreMesh(**_MESH),
        out_type=jax.ShapeDtypeStruct((NC * NA, HID), jnp.float32),
        scratch_types=[
            pltpu.VMEM((RPW, M), jnp.int32),
            pltpu.VMEM((CH, HID), jnp.float32),
            pltpu.VMEM((NPS, HID), jnp.float32),
            pltpu.VMEM_SHARED((NA, HID), jnp.float32),
            pltpu.SemaphoreType.DMA,
        ],
        compiler_params=pltpu.CompilerParams(use_tc_tiling_on_sc=False),
    )(_sc_scatter_body)


BE = 4096


def _msg_body(xj_ref, ea_ref, wf_ref, r_ref, out_ref):
    xj = xj_ref[...]
    ea = ea_ref[...]
    acc = None
    for g in range(4):
        tg = jnp.dot(xj, wf_ref[:, 128 * g:128 * (g + 1)],
                     preferred_element_type=jnp.float32)
        ug = jnp.dot(ea, r_ref[:, 128 * g:128 * (g + 1)],
                     preferred_element_type=jnp.float32)
        vg = tg * ug
        acc = vg if acc is None else acc + vg
    tb = jnp.dot(xj, wf_ref[:, 512:544], preferred_element_type=jnp.float32)
    ub = jnp.dot(ea, r_ref[:, 512:544], preferred_element_type=jnp.float32)
    b = acc[:, 0:64] + acc[:, 64:128]
    out_ref[...] = b[:, 0:32] + b[:, 32:64] + tb * ub


def _tc_msg(x_j, ea_aug, wfull, r_m):
    return pl.pallas_call(
        _msg_body,
        grid=(EP // BE,),
        in_specs=[
            pl.BlockSpec((BE, IN), lambda i: (i, 0)),
            pl.BlockSpec((BE, KP1), lambda i: (i, 0)),
            pl.BlockSpec((IN, KW), lambda i: (0, 0)),
            pl.BlockSpec((KP1, KW), lambda i: (0, 0)),
        ],
        out_specs=pl.BlockSpec((BE, HID), lambda i: (i, 0)),
        out_shape=jax.ShapeDtypeStruct((EP, HID), jnp.float32),
    )(x_j, ea_aug, wfull, r_m)


def _epi_body(p0_ref, p1_ref, x_ref, b_ref, root_ref, cb_ref, g_ref, be_ref,
              sw_ref, sb_ref, tw_ref, tb_ref, out_ref):
    h = (p0_ref[...] + p1_ref[...]
         + jnp.dot(x_ref[...], root_ref[...],
                   preferred_element_type=jnp.float32)
         + cb_ref[...])
    mean = jnp.mean(h, axis=0, keepdims=True)
    var = jnp.mean(jnp.square(h - mean), axis=0, keepdims=True)
    hn = (h - mean) * lax.rsqrt(var + 1e-5) * g_ref[...] + be_ref[...]
    hn = jnp.maximum(hn, 0.0)
    s_x = jnp.dot(hn, sw_ref[...],
                  preferred_element_type=jnp.float32) + sb_ref[...]
    t_x = jnp.dot(hn, tw_ref[...],
                  preferred_element_type=jnp.float32) + tb_ref[...]
    s_x = jnp.clip(s_x, -30.0, 30.0)
    f_x = jnp.tanh(t_x) / (1.0 + jnp.exp(s_x))
    gid = lax.broadcasted_iota(jnp.int32, (N, G), 1)
    mask = (b_ref[...] == gid).astype(jnp.float32)
    out_ref[...] = lax.dot_general(mask, f_x, (((0,), (0,)), ((), ())),
                                   preferred_element_type=jnp.float32)


def _tc_epilogue(p0, p1, x, batch2d, root, cb, gamma, beta, sw, sb, tw, tb):
    full = lambda s: pl.BlockSpec(s, lambda: (0,) * len(s))
    return pl.pallas_call(
        _epi_body,
        in_specs=[
            full((N, HID)), full((N, HID)), full((N, IN)), full((N, 1)),
            full((IN, HID)), full((1, HID)), full((1, HID)), full((1, HID)),
            full((HID, OUT)), full((1, OUT)), full((HID, OUT)),
            full((1, OUT)),
        ],
        out_specs=full((G, OUT)),
        out_shape=jax.ShapeDtypeStruct((G, OUT), jnp.float32),
    )(p0, p1, x, batch2d, root, cb, gamma, beta, sw, sb, tw, tb)


def kernel(x, edge_index, edge_attr, batch, edge_batch,
           nn_W, nn_b, root, conv_bias, bn_gamma, bn_beta,
           s_W, s_b, t_W, t_b):
    pad = EP - E
    src = jnp.concatenate(
        [edge_index[0].astype(jnp.int32), jnp.zeros((pad,), jnp.int32)]
    ).reshape(ROWS, M)
    dst = jnp.concatenate(
        [edge_index[1].astype(jnp.int32), jnp.full((pad,), N, jnp.int32)]
    ).reshape(ROWS, M)

    x_j = _make_sc_gather()(x.astype(jnp.bfloat16), src)

    ea_aug = jnp.concatenate(
        [edge_attr, jnp.ones((E, 1), jnp.float32)], axis=1)
    ea_aug = jnp.concatenate(
        [ea_aug, jnp.zeros((pad, KP1), jnp.float32)], axis=0
    ).astype(jnp.bfloat16)
    wfull = jnp.concatenate(
        [nn_W.reshape(IN, HID, EDGE).transpose(0, 2, 1).reshape(IN, EDGE * HID),
         nn_b.reshape(IN, HID)], axis=1)
    msg = _tc_msg(x_j, ea_aug, wfull.astype(jnp.bfloat16),
                  jnp.asarray(_R_NP, dtype=jnp.bfloat16))

    partials = _make_sc_scatter()(
        msg, dst, jnp.zeros((NPS, HID), jnp.float32))

    out = _tc_epilogue(
        partials[:N], partials[NA:NA + N], x,
        batch.astype(jnp.int32).reshape(N, 1),
        root, conv_bias.reshape(1, HID),
        bn_gamma.reshape(1, HID), bn_beta.reshape(1, HID),
        s_W.T, s_b.reshape(1, OUT), t_W.T, t_b.reshape(1, OUT))
    return out
